# baseline (speedup 1.0000x reference)
import functools

import jax
import jax.numpy as jnp
from jax import lax
from jax.experimental import pallas as pl
from jax.experimental.pallas import tpu as pltpu
from jax.experimental.pallas import tpu_sc as plsc

K = 32
B = 4
S = 4096
T = B * S
LANES = 16

NUM_CORES = 2
NUM_SUBCORES = 16
NW = NUM_CORES * NUM_SUBCORES
TOK_PER_W = T // NW
CHUNK = 128
N_CHUNKS = TOK_PER_W // CHUNK
GROUPS = CHUNK // LANES
PADR = K + 1
PADC = LANES + 1


def _sc_body(vals_hbm, dist_hbm, out_hbm, vals_v, dist_v, stage_v, pad_v,
             cnt_v):
    wid = lax.axis_index("s") * NUM_CORES + lax.axis_index("c")
    base = wid * TOK_PER_W
    lane_iota = lax.iota(jnp.int32, LANES)

    def chunk_body(ci, carry):
        tok0 = base + ci * CHUNK
        pltpu.sync_copy(vals_hbm.at[pl.ds(tok0, CHUNK)], vals_v)
        pltpu.sync_copy(dist_hbm.at[pl.ds(tok0, CHUNK)], dist_v)

        def group_body(g, carry):
            r0 = g * LANES
            for t in range(LANES):
                for h in range(2):
                    sl = pl.ds(h * LANES, LANES)
                    pad_v[t, sl] = vals_v[r0 + t, sl]
                    stage_v[r0 + t, sl] = dist_v[r0 + t, sl]
            cols = [
                plsc.load_gather(
                    pad_v, [lane_iota, jnp.full((LANES,), j, jnp.int32)])
                for j in range(K)
            ]
            count = jnp.zeros((LANES,), jnp.int32)
            for j in range(K):
                vj = cols[j]
                terms = [cols[l] == vj for l in range(j)]
                while len(terms) > 1:
                    nxt = []
                    for i in range(0, len(terms) - 1, 2):
                        nxt.append(terms[i] | terms[i + 1])
                    if len(terms) % 2:
                        nxt.append(terms[-1])
                    terms = nxt
                new = vj != 0
                if terms:
                    new = new & jnp.logical_not(terms[0])
                count = count + new.astype(jnp.int32)
                cnt_v[j, pl.ds(0, LANES)] = count.astype(jnp.float32)
            for t in range(LANES):
                for h in range(2):
                    kk = h * LANES + lane_iota
                    cvec = plsc.load_gather(
                        cnt_v, [kk, jnp.full((LANES,), t, jnp.int32)])
                    stage_v[r0 + t, pl.ds(K + h * LANES, LANES)] = cvec
            return carry

        lax.fori_loop(0, GROUPS, group_body, 0)
        pltpu.sync_copy(stage_v, out_hbm.at[pl.ds(tok0, CHUNK)])
        return carry

    lax.fori_loop(0, N_CHUNKS, chunk_body, 0)


@functools.partial(jax.jit, static_argnames=())
def kernel(vals, distances):
    vals2 = vals.reshape(T, K)
    dist2 = distances.reshape(T, K)
    mesh = plsc.VectorSubcoreMesh(
        core_axis_name="c", subcore_axis_name="s",
        num_cores=NUM_CORES, num_subcores=NUM_SUBCORES)
    out = pl.kernel(
        _sc_body,
        out_type=jax.ShapeDtypeStruct((T, 2 * K), jnp.float32),
        mesh=mesh,
        scratch_types=[
            pltpu.VMEM((CHUNK, K), jnp.int32),
            pltpu.VMEM((CHUNK, K), jnp.float32),
            pltpu.VMEM((CHUNK, 2 * K), jnp.float32),
            pltpu.VMEM((LANES, PADR), jnp.int32),
            pltpu.VMEM((K, PADC), jnp.float32),
        ],
        compiler_params=pltpu.CompilerParams(needs_layout_passes=False),
    )(vals2, dist2)
    return out.reshape(B, S, 2 * K)


# speedup vs baseline: 1.1338x; 1.1338x over previous
import functools

import jax
import jax.numpy as jnp
from jax import lax
from jax.experimental import pallas as pl
from jax.experimental.pallas import tpu as pltpu
from jax.experimental.pallas import tpu_sc as plsc

K = 32
B = 4
S = 4096
T = B * S
LANES = 16
GL = 2 * LANES

NUM_CORES = 2
NUM_SUBCORES = 16
NW = NUM_CORES * NUM_SUBCORES
TOK_PER_W = T // NW
CHUNK = 128
N_CHUNKS = TOK_PER_W // CHUNK
GROUPS = CHUNK // GL


def _sc_body(vals_hbm, dist_hbm, out_hbm, vals_v, dist_v, stage_v):
    wid = lax.axis_index("s") * NUM_CORES + lax.axis_index("c")
    base = wid * TOK_PER_W
    lane_iota = lax.iota(jnp.int32, LANES)

    def chunk_body(ci, carry):
        tok0 = base + ci * CHUNK
        pltpu.sync_copy(vals_hbm.at[pl.ds(tok0, CHUNK)], vals_v)
        pltpu.sync_copy(dist_hbm.at[pl.ds(tok0, CHUNK)], dist_v)

        def group_body(g, carry):
            r0 = g * GL
            rows_lo = r0 + lane_iota
            rows_hi = r0 + LANES + lane_iota
            for t in range(GL):
                for h in range(2):
                    sl = pl.ds(h * LANES, LANES)
                    stage_v[r0 + t, sl] = dist_v[r0 + t, sl]
            cols = []
            for j in range(K):
                cj = jnp.full((LANES,), j, jnp.int32)
                lo = plsc.load_gather(vals_v, [rows_lo, cj])
                hi = plsc.load_gather(vals_v, [rows_hi, cj])
                cols.append(lo | (hi << 16))
            count = jnp.zeros((LANES,), jnp.int32)
            for j in range(K):
                vj = cols[j]
                terms = [plsc.bitcast(vl ^ vj, jnp.uint16)
                         for vl in cols[:j]]
                terms.append(plsc.bitcast(vj, jnp.uint16))
                while len(terms) > 1:
                    nxt = []
                    for i in range(0, len(terms) - 1, 2):
                        nxt.append(jnp.minimum(terms[i], terms[i + 1]))
                    if len(terms) % 2:
                        nxt.append(terms[-1])
                    terms = nxt
                z32 = plsc.bitcast(terms[0], jnp.int32)
                nlo = ((z32 & 0xFFFF) != 0).astype(jnp.int32)
                nhi = ((z32 >> 16) != 0).astype(jnp.int32)
                count = count + nlo + (nhi << 16)
                colj = jnp.full((LANES,), K + j, jnp.int32)
                plsc.store_scatter(
                    stage_v, [rows_lo, colj],
                    (count & 0xFFFF).astype(jnp.float32))
                plsc.store_scatter(
                    stage_v, [rows_hi, colj],
                    (count >> 16).astype(jnp.float32))
            return carry

        lax.fori_loop(0, GROUPS, group_body, 0)
        pltpu.sync_copy(stage_v, out_hbm.at[pl.ds(tok0, CHUNK)])
        return carry

    lax.fori_loop(0, N_CHUNKS, chunk_body, 0)


@functools.partial(jax.jit, static_argnames=())
def kernel(vals, distances):
    vals2 = vals.reshape(T, K)
    dist2 = distances.reshape(T, K)
    mesh = plsc.VectorSubcoreMesh(
        core_axis_name="c", subcore_axis_name="s",
        num_cores=NUM_CORES, num_subcores=NUM_SUBCORES)
    out = pl.kernel(
        _sc_body,
        out_type=jax.ShapeDtypeStruct((T, 2 * K), jnp.float32),
        mesh=mesh,
        scratch_types=[
            pltpu.VMEM((CHUNK, K), jnp.int32),
            pltpu.VMEM((CHUNK, K), jnp.float32),
            pltpu.VMEM((CHUNK, 2 * K), jnp.float32),
        ],
        compiler_params=pltpu.CompilerParams(needs_layout_passes=False),
    )(vals2, dist2)
    return out.reshape(B, S, 2 * K)
